# Initial kernel scaffold; baseline (speedup 1.0000x reference)
import jax
import jax.numpy as jnp
from jax.experimental import pallas as pl


def kernel(gene_x, cpg_x, mirna_x, gene_batch, cpg_batch, mirna_batch, mrna_W, mrna_b, cnv_W, cnv_b):
    raise NotImplementedError("write your pallas kernel here")


# speedup vs baseline: 8.6786x; 8.6786x over previous
import functools

import jax
import jax.numpy as jnp
from jax import lax
from jax.experimental import pallas as pl
from jax.experimental.pallas import tpu as pltpu
from jax.experimental.pallas import tpu_sc as plsc

NUM_SEG = 16
H = 128
LANES = 16
GROUPS = H // LANES
CHUNK = 256


def _make_sc_pool(n_gene, n_cpg, n_mirna):
    try:
        info = plsc.get_sparse_core_info()
        nc, ns = info.num_cores, info.num_subcores
    except Exception:
        nc, ns = 2, 16
    nw = nc * ns
    sizes = (n_gene, n_cpg, n_mirna)

    mesh = plsc.VectorSubcoreMesh(core_axis_name="c", subcore_axis_name="s",
                                  num_cores=nc, num_subcores=ns)
    out_types = [jax.ShapeDtypeStruct((nw, NUM_SEG, H), jnp.float32)
                 for _ in sizes]
    scratch = [
        pltpu.VMEM((CHUNK, H), jnp.float32),
        pltpu.VMEM((32,), jnp.int32),
        pltpu.VMEM((NUM_SEG, H), jnp.float32),
    ]

    @functools.partial(pl.kernel, mesh=mesh, out_type=out_types,
                       scratch_types=scratch)
    def sc_pool(x0, x1, x2, b0, b1, b2, o0, o1, o2, buf, bnd, acc):
        wid = lax.axis_index("c") * ns + lax.axis_index("s")
        zero = jnp.zeros((LANES,), jnp.float32)
        iota = lax.iota(jnp.int32, LANES)

        for x_hbm, bnd_hbm, out_hbm, n in (
                (x0, b0, o0, sizes[0]), (x1, b1, o1, sizes[1]),
                (x2, b2, o2, sizes[2])):
            per = -(-(-(-n // nw)) // 8) * 8
            lo = wid * per
            hi = jnp.minimum(lo + per, n)
            pltpu.sync_copy(bnd_hbm, bnd)

            def zbody(i, _):
                for j in range(GROUPS):
                    acc[i, pl.ds(j * LANES, LANES)] = zero
                return 0
            lax.fori_loop(0, NUM_SEG, zbody, 0)

            v0 = bnd[pl.ds(0, LANES)]
            v1 = bnd[pl.ds(LANES, LANES)]
            bs = [v0[s] if s < LANES else v1[s - LANES]
                  for s in range(NUM_SEG + 1)]

            nck = -(-per // CHUNK)

            def cbody(ck, _):
                cstart = lo + ck * CHUNK
                cend = jnp.minimum(cstart + CHUNK, hi)
                base = pl.multiple_of(jnp.minimum(cstart, n - CHUNK), 8)
                pltpu.sync_copy(x_hbm.at[pl.ds(base, CHUNK)], buf)
                for s in range(NUM_SEG):
                    r0 = jnp.maximum(bs[s], cstart)
                    r1 = jnp.minimum(bs[s + 1], cend)

                    @pl.when(r1 > r0)
                    def _():
                        def rbody(r, vs):
                            rl = r - base
                            return tuple(
                                vs[j] + buf[rl, pl.ds(j * LANES, LANES)]
                                for j in range(GROUPS))
                        vs = lax.fori_loop(
                            r0, r1, rbody,
                            tuple(zero for _ in range(GROUPS)))
                        for j in range(GROUPS):
                            acc[s, pl.ds(j * LANES, LANES)] += vs[j]
                return 0

            lax.fori_loop(0, nck, cbody, 0)
            pltpu.sync_copy(acc, out_hbm.at[wid])

    return sc_pool, nw


def _fin_body(pg, pc, pm, cg, cc, cm, wm, bm, wc, bc,
              o_mrna, o_cnv, o_dna, o_mir):
    gsum = jnp.sum(pg[...], axis=0)
    g = gsum / jnp.maximum(cg[...], 1.0)
    dn = (((1,), (1,)), ((), ()))
    o_mrna[...] = lax.dot_general(g, wm[...], dn,
                                  preferred_element_type=jnp.float32) + bm[...]
    o_cnv[...] = lax.dot_general(g, wc[...], dn,
                                 preferred_element_type=jnp.float32) + bc[...]
    o_dna[...] = jnp.sum(pc[...], axis=0) / jnp.maximum(cc[...], 1.0)
    o_mir[...] = jnp.sum(pm[...], axis=0) / jnp.maximum(cm[...], 1.0)


def kernel(gene_x, cpg_x, mirna_x, gene_batch, cpg_batch, mirna_batch,
           mrna_W, mrna_b, cnv_W, cnv_b):
    seg_ids = jnp.arange(NUM_SEG + 1, dtype=jnp.int32)

    def bounds_of(batch, n):
        b = jnp.searchsorted(batch, seg_ids, side="left").astype(jnp.int32)
        return jnp.concatenate([b, jnp.full((32 - (NUM_SEG + 1),), n,
                                            dtype=jnp.int32)])

    gb = bounds_of(gene_batch, gene_x.shape[0])
    cb = bounds_of(cpg_batch, cpg_x.shape[0])
    mb = bounds_of(mirna_batch, mirna_x.shape[0])

    sc_pool, nw = _make_sc_pool(gene_x.shape[0], cpg_x.shape[0],
                                mirna_x.shape[0])
    pg, pc, pm = sc_pool(gene_x, cpg_x, mirna_x, gb, cb, mb)

    def counts_of(b):
        c = (b[1:NUM_SEG + 1] - b[:NUM_SEG]).astype(jnp.float32)
        return jnp.broadcast_to(c[:, None], (NUM_SEG, H))

    outs = pl.pallas_call(
        _fin_body,
        out_shape=[jax.ShapeDtypeStruct((NUM_SEG, H), jnp.float32)] * 4,
    )(pg, pc, pm, counts_of(gb), counts_of(cb), counts_of(mb),
      mrna_W, mrna_b.reshape(1, H), cnv_W, cnv_b.reshape(1, H))
    return tuple(outs)


# speedup vs baseline: 12.4972x; 1.4400x over previous
import functools

import jax
import jax.numpy as jnp
from jax import lax
from jax.experimental import pallas as pl
from jax.experimental.pallas import tpu as pltpu
from jax.experimental.pallas import tpu_sc as plsc

NUM_SEG = 16
H = 128
LANES = 16
GROUPS = H // LANES
CHUNK = 256


def _make_sc_pool(n_gene, n_cpg, n_mirna):
    try:
        info = plsc.get_sparse_core_info()
        nc, ns = info.num_cores, info.num_subcores
    except Exception:
        nc, ns = 2, 16
    nw = nc * ns
    sizes = (n_gene, n_cpg, n_mirna)

    mesh = plsc.VectorSubcoreMesh(core_axis_name="c", subcore_axis_name="s",
                                  num_cores=nc, num_subcores=ns)
    out_types = [jax.ShapeDtypeStruct((nw, NUM_SEG, H), jnp.float32)
                 for _ in sizes]
    scratch = [
        pltpu.VMEM((CHUNK, H), jnp.float32),
        pltpu.VMEM((CHUNK, H), jnp.float32),
        pltpu.VMEM((32,), jnp.int32),
        pltpu.VMEM((NUM_SEG, H), jnp.float32),
        pltpu.SemaphoreType.DMA,
        pltpu.SemaphoreType.DMA,
    ]

    @functools.partial(pl.kernel, mesh=mesh, out_type=out_types,
                       scratch_types=scratch)
    def sc_pool(x0, x1, x2, b0, b1, b2, o0, o1, o2,
                bufa, bufb, bnd, acc, sema, semb):
        bufs = (bufa, bufb)
        sems = (sema, semb)
        wid = lax.axis_index("c") * ns + lax.axis_index("s")
        zero = jnp.zeros((LANES,), jnp.float32)
        iota = lax.iota(jnp.int32, LANES)

        for x_hbm, bnd_hbm, out_hbm, n in (
                (x0, b0, o0, sizes[0]), (x1, b1, o1, sizes[1]),
                (x2, b2, o2, sizes[2])):
            per = -(-(-(-n // nw)) // 8) * 8
            lo = wid * per
            hi = jnp.minimum(lo + per, n)
            pltpu.sync_copy(bnd_hbm, bnd)

            def zbody(i, _):
                for j in range(GROUPS):
                    acc[i, pl.ds(j * LANES, LANES)] = zero
                return 0
            lax.fori_loop(0, NUM_SEG, zbody, 0)

            v0 = bnd[pl.ds(0, LANES)]
            v1 = bnd[pl.ds(LANES, LANES)]
            bs = [v0[s] if s < LANES else v1[s - LANES]
                  for s in range(NUM_SEG + 1)]

            nck = -(-per // CHUNK)

            def dma_base(ck):
                return pl.multiple_of(
                    jnp.minimum(lo + ck * CHUNK, n - CHUNK), 8)

            def dma(ck, b):
                return pltpu.make_async_copy(
                    x_hbm.at[pl.ds(dma_base(ck), CHUNK)], bufs[b], sems[b])

            def process(ck, b):
                cstart = lo + ck * CHUNK
                cend = jnp.minimum(cstart + CHUNK, hi)
                base = dma_base(ck)
                buf = bufs[b]
                for s in range(NUM_SEG):
                    r0 = jnp.maximum(bs[s], cstart)
                    r1 = jnp.minimum(bs[s + 1], cend)

                    @pl.when(r1 > r0)
                    def _():
                        def rbody(r, vs):
                            rl = r - base
                            return tuple(
                                vs[j] + buf[rl, pl.ds(j * LANES, LANES)]
                                for j in range(GROUPS))
                        vs = lax.fori_loop(
                            r0, r1, rbody,
                            tuple(zero for _ in range(GROUPS)))
                        for j in range(GROUPS):
                            acc[s, pl.ds(j * LANES, LANES)] += vs[j]

            for b in range(min(2, nck)):
                dma(b, b).start()

            def pbody(p, _):
                for b in range(2):
                    ck = p * 2 + b

                    @pl.when(ck < nck)
                    def _():
                        dma(ck, b).wait()
                        process(ck, b)

                        @pl.when(ck + 2 < nck)
                        def _():
                            dma(ck + 2, b).start()
                return 0

            lax.fori_loop(0, -(-nck // 2), pbody, 0)
            pltpu.sync_copy(acc, out_hbm.at[wid])

    return sc_pool, nw


def _fin_body(pg, pc, pm, cg, cc, cm, wm, bm, wc, bc,
              o_mrna, o_cnv, o_dna, o_mir):
    gsum = jnp.sum(pg[...], axis=0)
    g = gsum / jnp.maximum(cg[...], 1.0)
    dn = (((1,), (1,)), ((), ()))
    o_mrna[...] = lax.dot_general(g, wm[...], dn,
                                  preferred_element_type=jnp.float32) + bm[...]
    o_cnv[...] = lax.dot_general(g, wc[...], dn,
                                 preferred_element_type=jnp.float32) + bc[...]
    o_dna[...] = jnp.sum(pc[...], axis=0) / jnp.maximum(cc[...], 1.0)
    o_mir[...] = jnp.sum(pm[...], axis=0) / jnp.maximum(cm[...], 1.0)


def kernel(gene_x, cpg_x, mirna_x, gene_batch, cpg_batch, mirna_batch,
           mrna_W, mrna_b, cnv_W, cnv_b):
    seg_ids = jnp.arange(NUM_SEG + 1, dtype=jnp.int32)

    def bounds_of(batch, n):
        b = jnp.searchsorted(batch, seg_ids, side="left").astype(jnp.int32)
        return jnp.concatenate([b, jnp.full((32 - (NUM_SEG + 1),), n,
                                            dtype=jnp.int32)])

    gb = bounds_of(gene_batch, gene_x.shape[0])
    cb = bounds_of(cpg_batch, cpg_x.shape[0])
    mb = bounds_of(mirna_batch, mirna_x.shape[0])

    sc_pool, nw = _make_sc_pool(gene_x.shape[0], cpg_x.shape[0],
                                mirna_x.shape[0])
    pg, pc, pm = sc_pool(gene_x, cpg_x, mirna_x, gb, cb, mb)

    def counts_of(b):
        c = (b[1:NUM_SEG + 1] - b[:NUM_SEG]).astype(jnp.float32)
        return jnp.broadcast_to(c[:, None], (NUM_SEG, H))

    outs = pl.pallas_call(
        _fin_body,
        out_shape=[jax.ShapeDtypeStruct((NUM_SEG, H), jnp.float32)] * 4,
    )(pg, pc, pm, counts_of(gb), counts_of(cb), counts_of(mb),
      mrna_W, mrna_b.reshape(1, H), cnv_W, cnv_b.reshape(1, H))
    return tuple(outs)


# speedup vs baseline: 17.4187x; 1.3938x over previous
import functools

import jax
import jax.numpy as jnp
from jax import lax
from jax.experimental import pallas as pl
from jax.experimental.pallas import tpu as pltpu
from jax.experimental.pallas import tpu_sc as plsc

NUM_SEG = 16
H = 128
LANES = 16
GROUPS = H // LANES
CHUNK = 256


def _make_sc_pool(n_gene, n_cpg, n_mirna):
    try:
        info = plsc.get_sparse_core_info()
        nc, ns = info.num_cores, info.num_subcores
    except Exception:
        nc, ns = 2, 16
    nw = nc * ns
    sizes = (n_gene, n_cpg, n_mirna)

    mesh = plsc.VectorSubcoreMesh(core_axis_name="c", subcore_axis_name="s",
                                  num_cores=nc, num_subcores=ns)
    out_types = [jax.ShapeDtypeStruct((nw, NUM_SEG, H), jnp.float32)
                 for _ in sizes]
    scratch = [
        pltpu.VMEM((CHUNK, H), jnp.float32),
        pltpu.VMEM((CHUNK, H), jnp.float32),
        pltpu.VMEM((32,), jnp.int32),
        pltpu.VMEM((NUM_SEG, H), jnp.float32),
        pltpu.SemaphoreType.DMA,
        pltpu.SemaphoreType.DMA,
    ]

    @functools.partial(pl.kernel, mesh=mesh, out_type=out_types,
                       scratch_types=scratch)
    def sc_pool(x0, x1, x2, b0, b1, b2, o0, o1, o2,
                bufa, bufb, bnd, acc, sema, semb):
        bufs = (bufa, bufb)
        sems = (sema, semb)
        wid = lax.axis_index("c") * ns + lax.axis_index("s")
        zero = jnp.zeros((LANES,), jnp.float32)
        iota = lax.iota(jnp.int32, LANES)

        for x_hbm, bnd_hbm, out_hbm, n in (
                (x0, b0, o0, sizes[0]), (x1, b1, o1, sizes[1]),
                (x2, b2, o2, sizes[2])):
            per = -(-(-(-n // nw)) // 8) * 8
            lo = wid * per
            hi = jnp.minimum(lo + per, n)
            pltpu.sync_copy(bnd_hbm, bnd)

            def zbody(i, _):
                for j in range(GROUPS):
                    acc[i, pl.ds(j * LANES, LANES)] = zero
                return 0
            lax.fori_loop(0, NUM_SEG, zbody, 0)

            v0 = bnd[pl.ds(0, LANES)]
            v1 = bnd[pl.ds(LANES, LANES)]
            bs = [v0[s] if s < LANES else v1[s - LANES]
                  for s in range(NUM_SEG + 1)]

            nck = -(-per // CHUNK)

            def dma_base(ck):
                return pl.multiple_of(
                    jnp.minimum(lo + ck * CHUNK, n - CHUNK), 8)

            def dma(ck, b):
                return pltpu.make_async_copy(
                    x_hbm.at[pl.ds(dma_base(ck), CHUNK)], bufs[b], sems[b])

            def process(ck, b):
                cstart = lo + ck * CHUNK
                cend = jnp.minimum(cstart + CHUNK, hi)
                base = dma_base(ck)
                buf = bufs[b]
                for s in range(NUM_SEG):
                    r0 = jnp.maximum(bs[s], cstart)
                    r1 = jnp.minimum(bs[s + 1], cend)

                    @pl.when(r1 > r0)
                    def _():
                        def rbody(r, vs):
                            rl = r - base
                            return tuple(
                                vs[j] + buf[rl, pl.ds(j * LANES, LANES)]
                                for j in range(GROUPS))
                        vs = lax.fori_loop(
                            r0, r1, rbody,
                            tuple(zero for _ in range(GROUPS)))
                        for j in range(GROUPS):
                            acc[s, pl.ds(j * LANES, LANES)] += vs[j]

            for b in range(min(2, nck)):
                dma(b, b).start()

            def pbody(p, _):
                for b in range(2):
                    ck = p * 2 + b

                    @pl.when(ck < nck)
                    def _():
                        dma(ck, b).wait()
                        process(ck, b)

                        @pl.when(ck + 2 < nck)
                        def _():
                            dma(ck + 2, b).start()
                return 0

            lax.fori_loop(0, -(-nck // 2), pbody, 0)
            pltpu.sync_copy(acc, out_hbm.at[wid])

    return sc_pool, nw


def _bounds_body(gids, cids, mids, gout, cout, mout):
    lane = lax.broadcasted_iota(jnp.int32, (1, 32), 1)
    for ids_ref, out_ref in ((gids, gout), (cids, cout), (mids, mout)):
        data = ids_ref[...]
        b = jnp.zeros((1, 32), jnp.float32)
        for s in range(NUM_SEG):
            cnt = jnp.sum(jnp.where(data == s, 1.0, 0.0))
            b = b + jnp.where(lane > s, cnt, 0.0)
        out_ref[...] = b.astype(jnp.int32)


def _fin_body(pg, pc, pm, cg, cc, cm, wm, bm, wc, bc,
              o_mrna, o_cnv, o_dna, o_mir):
    gsum = jnp.sum(pg[...], axis=0)
    g = gsum / jnp.maximum(cg[...], 1.0)
    dn = (((1,), (1,)), ((), ()))
    o_mrna[...] = lax.dot_general(g, wm[...], dn,
                                  preferred_element_type=jnp.float32) + bm[...]
    o_cnv[...] = lax.dot_general(g, wc[...], dn,
                                 preferred_element_type=jnp.float32) + bc[...]
    o_dna[...] = jnp.sum(pc[...], axis=0) / jnp.maximum(cc[...], 1.0)
    o_mir[...] = jnp.sum(pm[...], axis=0) / jnp.maximum(cm[...], 1.0)


def kernel(gene_x, cpg_x, mirna_x, gene_batch, cpg_batch, mirna_batch,
           mrna_W, mrna_b, cnv_W, cnv_b):
    gb, cb, mb = pl.pallas_call(
        _bounds_body,
        out_shape=[jax.ShapeDtypeStruct((1, 32), jnp.int32)] * 3,
    )(gene_batch.reshape(-1, H), cpg_batch.reshape(-1, H),
      mirna_batch.reshape(-1, H))
    gb, cb, mb = gb.reshape(32), cb.reshape(32), mb.reshape(32)

    sc_pool, nw = _make_sc_pool(gene_x.shape[0], cpg_x.shape[0],
                                mirna_x.shape[0])
    pg, pc, pm = sc_pool(gene_x, cpg_x, mirna_x, gb, cb, mb)

    def counts_of(b):
        c = (b[1:NUM_SEG + 1] - b[:NUM_SEG]).astype(jnp.float32)
        return jnp.broadcast_to(c[:, None], (NUM_SEG, H))

    outs = pl.pallas_call(
        _fin_body,
        out_shape=[jax.ShapeDtypeStruct((NUM_SEG, H), jnp.float32)] * 4,
    )(pg, pc, pm, counts_of(gb), counts_of(cb), counts_of(mb),
      mrna_W, mrna_b.reshape(1, H), cnv_W, cnv_b.reshape(1, H))
    return tuple(outs)
